# Initial kernel scaffold; baseline (speedup 1.0000x reference)
import jax
import jax.numpy as jnp
from jax.experimental import pallas as pl


def kernel(x, batch, W1, b1, W2, b2):
    raise NotImplementedError("write your pallas kernel here")


# speedup vs baseline: 7.5449x; 7.5449x over previous
import jax
import jax.numpy as jnp
from jax.experimental import pallas as pl
from jax.experimental.pallas import tpu as pltpu

_S = 512
_B = 1000


def _body(x_ref, b_ref, w1_ref, b1_ref, w2_ref, out_ref, m_ref, d_ref, acc_ref):
    i = pl.program_id(0)
    nb = pl.num_programs(0)
    neg = jnp.float32(-jnp.inf)

    @pl.when(i == 0)
    def _init():
        m_ref[...] = jnp.full(m_ref.shape, neg, jnp.float32)
        d_ref[...] = jnp.zeros(d_ref.shape, jnp.float32)
        acc_ref[...] = jnp.zeros(acc_ref.shape, jnp.float32)

    xb = x_ref[...]
    h = jnp.maximum(
        jnp.dot(xb, w1_ref[...], preferred_element_type=jnp.float32)
        + b1_ref[...], 0.0)
    s = jnp.sum(h * w2_ref[...], axis=1, keepdims=True)

    seg = b_ref[0]
    ids = jax.lax.broadcasted_iota(jnp.int32, (_B, _S), 1)
    mask = seg == ids

    bm = jnp.max(jnp.where(mask, s, neg), axis=0, keepdims=True)
    m_old = m_ref[...]
    m_new = jnp.maximum(m_old, bm)
    scale = jnp.where(m_old == neg, 0.0, jnp.exp(m_old - m_new))

    m_per = jnp.max(jnp.where(mask, m_new, neg), axis=1, keepdims=True)
    p = jnp.exp(s - m_per)
    P = jnp.where(mask, p, 0.0)

    d_ref[...] = d_ref[...] * scale + jnp.sum(P, axis=0, keepdims=True)
    acc_ref[...] = acc_ref[...] * scale + jax.lax.dot_general(
        xb, P, (((0,), (0,)), ((), ())),
        preferred_element_type=jnp.float32)
    m_ref[...] = m_new

    @pl.when(i == nb - 1)
    def _fin():
        d = d_ref[...]
        invd = jnp.where(d > 0, 1.0 / d, 0.0)
        r = jax.lax.broadcasted_iota(jnp.int32, (_S, _S), 0)
        c = jax.lax.broadcasted_iota(jnp.int32, (_S, _S), 1)
        dm = jnp.where(r == c, invd, 0.0)
        out_ref[...] = jax.lax.dot_general(
            dm, acc_ref[...], (((1,), (1,)), ((), ())),
            preferred_element_type=jnp.float32)


def kernel(x, batch, W1, b1, W2, b2):
    n, d = x.shape
    nb = n // _B
    batch3 = batch.astype(jnp.int32).reshape(nb, _B, 1)
    return pl.pallas_call(
        _body,
        grid=(nb,),
        in_specs=[
            pl.BlockSpec((_B, d), lambda i: (i, 0)),
            pl.BlockSpec((1, _B, 1), lambda i: (i, 0, 0)),
            pl.BlockSpec((d, d), lambda i: (0, 0)),
            pl.BlockSpec((1, d), lambda i: (0, 0)),
            pl.BlockSpec((1, d), lambda i: (0, 0)),
        ],
        out_specs=pl.BlockSpec((_S, d), lambda i: (0, 0)),
        out_shape=jax.ShapeDtypeStruct((_S, d), jnp.float32),
        scratch_shapes=[
            pltpu.VMEM((1, _S), jnp.float32),
            pltpu.VMEM((1, _S), jnp.float32),
            pltpu.VMEM((d, _S), jnp.float32),
        ],
    )(x, batch3, W1, b1.reshape(1, d), W2.reshape(1, d))
